# baseline (speedup 1.0000x reference)
import functools

import jax
import jax.numpy as jnp
from jax import lax
from jax.experimental import pallas as pl
from jax.experimental.pallas import tpu as pltpu
from jax.experimental.pallas import tpu_sc as plsc

_B = 1024
_D = 768
_K = 100000
_BK = 5000
_NFC = _BK // 128


def _mlp_body(pos, neg, w1, b1, w2, b2, w3, b3, w4, b4, out):
    x = jnp.concatenate([pos[...], neg[...]], axis=1)
    h = jnp.maximum(jnp.dot(x, w1[...], preferred_element_type=jnp.float32) + b1[...], 0.0)
    h = jnp.maximum(jnp.dot(h, w2[...], preferred_element_type=jnp.float32) + b2[...], 0.0)
    h = jnp.maximum(jnp.dot(h, w3[...], preferred_element_type=jnp.float32) + b3[...], 0.0)
    o = jnp.dot(h, w4[...], preferred_element_type=jnp.float32) + b4[...]
    n = jnp.sqrt(jnp.sum(o * o, axis=1, keepdims=True))
    out[...] = o / jnp.maximum(n, 1e-12)


def _mlp(pos, neg, w1, b1, w2, b2, w3, b3, w4, b4):
    return pl.pallas_call(
        _mlp_body,
        out_shape=jax.ShapeDtypeStruct((_B, _D), jnp.float32),
    )(pos, neg, w1, b1, w2, b2, w3, b3, w4, b4)


def _search_body(q, db, idx_out, best_v, best_i):
    i = pl.program_id(0)

    @pl.when(i == 0)
    def _():
        best_v[...] = jnp.full((_B, 1), -jnp.inf, jnp.float32)
        best_i[...] = jnp.zeros((_B, 1), jnp.int32)

    d = db[...]
    dsq = d * d
    n2 = lax.dot_general(dsq, jnp.ones((1, _D), jnp.float32), (((1,), (1,)), ((), ())),
                         preferred_element_type=jnp.float32)
    rinv = 1.0 / jnp.maximum(jnp.sqrt(n2), 1e-12)
    dn = d * rinv
    s = lax.dot_general(q[...], dn, (((1,), (1,)), ((), ())),
                        preferred_element_type=jnp.float32)

    lane128 = lax.broadcasted_iota(jnp.int32, (8, 128), 1)
    lane_tail = lax.broadcasted_iota(jnp.int32, (8, _BK - _NFC * 128), 1)

    def _comb(left, right):
        ma, ca = left
        mb, cb = right
        upd = mb > ma
        return jnp.where(upd, mb, ma), jnp.where(upd, cb, ca)

    for rt in range(_B // 8):
        b0 = rt * 8
        stack = []
        for c in range(_NFC):
            node = (s[b0:b0 + 8, c * 128:(c + 1) * 128],
                    jnp.full((8, 128), c, jnp.int32))
            lvl = 0
            while stack and stack[-1][0] == lvl:
                _, ma, ca = stack.pop()
                node = _comb((ma, ca), node)
                lvl += 1
            stack.append((lvl, node[0], node[1]))
        node = (stack[-1][1], stack[-1][2])
        for ent in reversed(stack[:-1]):
            node = _comb((ent[1], ent[2]), node)
        tm, tc = node
        pm = s[b0:b0 + 8, _NFC * 128:_BK]
        m_t = jnp.maximum(jnp.max(tm, axis=1, keepdims=True),
                          jnp.max(pm, axis=1, keepdims=True))
        a1 = jnp.min(jnp.where(tm == m_t, tc * 128 + lane128, _K),
                     axis=1, keepdims=True)
        a2 = jnp.min(jnp.where(pm == m_t, _NFC * 128 + lane_tail, _K),
                     axis=1, keepdims=True)
        a_t = jnp.minimum(a1, a2)
        bv = best_v[b0:b0 + 8, :]
        upd = m_t > bv
        best_v[b0:b0 + 8, :] = jnp.where(upd, m_t, bv)
        bi = best_i[b0:b0 + 8, :]
        best_i[b0:b0 + 8, :] = jnp.where(upd, a_t + i * _BK, bi)

    @pl.when(i == pl.num_programs(0) - 1)
    def _():
        idx_out[...] = best_i[...]


def _search(q, db):
    return pl.pallas_call(
        _search_body,
        grid=(_K // _BK,),
        in_specs=[
            pl.BlockSpec((_B, _D), lambda i: (0, 0)),
            pl.BlockSpec((_BK, _D), lambda i: (i, 0)),
        ],
        out_specs=pl.BlockSpec((_B, 1), lambda i: (0, 0)),
        out_shape=jax.ShapeDtypeStruct((_B, 1), jnp.int32),
        scratch_shapes=[
            pltpu.VMEM((_B, 1), jnp.float32),
            pltpu.VMEM((_B, 1), jnp.int32),
        ],
    )(q, db)


_NC = 2
_NS = 16
_NW = _NC * _NS
_BPW = _B // _NW


def _sc_gather_body(db_hbm, idx_hbm, out_hbm, idx_v, rows_v, sem):
    wid = lax.axis_index("s") * _NC + lax.axis_index("c")
    base = wid * _BPW
    pltpu.sync_copy(idx_hbm.at[pl.ds(base, _BPW)], idx_v)
    pltpu.async_copy(db_hbm.at[idx_v], rows_v, sem).wait()
    pltpu.sync_copy(rows_v, out_hbm.at[pl.ds(base, _BPW)])


def _sc_gather(db, idx):
    k = functools.partial(
        pl.kernel,
        out_type=jax.ShapeDtypeStruct((_B, _D), jnp.float32),
        mesh=plsc.VectorSubcoreMesh(core_axis_name="c", subcore_axis_name="s"),
        scratch_types=[
            pltpu.VMEM((_BPW,), jnp.int32),
            pltpu.VMEM((_BPW, _D), jnp.float32),
            pltpu.SemaphoreType.DMA,
        ],
    )(_sc_gather_body)
    return k(db, idx)


def _norm_body(x_ref, o_ref):
    x = x_ref[...]
    n = jnp.sqrt(jnp.sum(x * x, axis=1, keepdims=True))
    o_ref[...] = x / jnp.maximum(n, 1e-12)


def _normalize(x):
    return pl.pallas_call(
        _norm_body,
        out_shape=jax.ShapeDtypeStruct((_B, _D), jnp.float32),
    )(x)


def kernel(pos_emb, neg_emb, db_keys, W1, b1, W2, b2, W3, b3, W4, b4):
    out = _mlp(pos_emb, neg_emb,
               W1, b1.reshape(1, -1), W2, b2.reshape(1, -1),
               W3, b3.reshape(1, -1), W4, b4.reshape(1, -1))
    idx = _search(out, db_keys)
    retrieved = _sc_gather(db_keys, idx.reshape(_B))
    search_out = _normalize(retrieved)
    return (out, search_out)


# speedup vs baseline: 1.0059x; 1.0059x over previous
import functools

import jax
import jax.numpy as jnp
from jax import lax
from jax.experimental import pallas as pl
from jax.experimental.pallas import tpu as pltpu
from jax.experimental.pallas import tpu_sc as plsc

_B = 1024
_D = 768
_K = 100000
_BK = 5000
_NFC = _BK // 128
_SR = 16


def _mlp_body(pos, neg, w1, b1, w2, b2, w3, b3, w4, b4, out):
    x = jnp.concatenate([pos[...], neg[...]], axis=1)
    h = jnp.maximum(jnp.dot(x, w1[...], preferred_element_type=jnp.float32) + b1[...], 0.0)
    h = jnp.maximum(jnp.dot(h, w2[...], preferred_element_type=jnp.float32) + b2[...], 0.0)
    h = jnp.maximum(jnp.dot(h, w3[...], preferred_element_type=jnp.float32) + b3[...], 0.0)
    o = jnp.dot(h, w4[...], preferred_element_type=jnp.float32) + b4[...]
    n = jnp.sqrt(jnp.sum(o * o, axis=1, keepdims=True))
    out[...] = o / jnp.maximum(n, 1e-12)


def _mlp(pos, neg, w1, b1, w2, b2, w3, b3, w4, b4):
    return pl.pallas_call(
        _mlp_body,
        out_shape=jax.ShapeDtypeStruct((_B, _D), jnp.float32),
    )(pos, neg, w1, b1, w2, b2, w3, b3, w4, b4)


def _search_body(q, db, idx_out, best_v, best_i):
    i = pl.program_id(0)

    @pl.when(i == 0)
    def _():
        best_v[...] = jnp.full((_B, 1), -jnp.inf, jnp.float32)
        best_i[...] = jnp.zeros((_B, 1), jnp.int32)

    d = db[...]
    dsq = d * d
    n2 = lax.dot_general(dsq, jnp.ones((1, _D), jnp.float32), (((1,), (1,)), ((), ())),
                         preferred_element_type=jnp.float32)
    rinv = 1.0 / jnp.maximum(jnp.sqrt(n2), 1e-12)
    dn = d * rinv
    s = lax.dot_general(q[...], dn, (((1,), (1,)), ((), ())),
                        preferred_element_type=jnp.float32)

    lane128 = lax.broadcasted_iota(jnp.int32, (_SR, 128), 1)
    lane_tail = lax.broadcasted_iota(jnp.int32, (_SR, _BK - _NFC * 128), 1)

    def _comb(left, right):
        ma, ca = left
        mb, cb = right
        upd = mb > ma
        return jnp.where(upd, mb, ma), jnp.where(upd, cb, ca)

    for rt in range(_B // _SR):
        b0 = rt * _SR
        stack = []
        for c in range(_NFC):
            node = (s[b0:b0 + _SR, c * 128:(c + 1) * 128],
                    jnp.full((_SR, 128), c, jnp.int32))
            lvl = 0
            while stack and stack[-1][0] == lvl:
                _, ma, ca = stack.pop()
                node = _comb((ma, ca), node)
                lvl += 1
            stack.append((lvl, node[0], node[1]))
        node = (stack[-1][1], stack[-1][2])
        for ent in reversed(stack[:-1]):
            node = _comb((ent[1], ent[2]), node)
        tm, tc = node
        pm = s[b0:b0 + _SR, _NFC * 128:_BK]
        m_t = jnp.maximum(jnp.max(tm, axis=1, keepdims=True),
                          jnp.max(pm, axis=1, keepdims=True))
        a1 = jnp.min(jnp.where(tm == m_t, tc * 128 + lane128, _K),
                     axis=1, keepdims=True)
        a2 = jnp.min(jnp.where(pm == m_t, _NFC * 128 + lane_tail, _K),
                     axis=1, keepdims=True)
        a_t = jnp.minimum(a1, a2)
        bv = best_v[b0:b0 + _SR, :]
        upd = m_t > bv
        best_v[b0:b0 + _SR, :] = jnp.where(upd, m_t, bv)
        bi = best_i[b0:b0 + _SR, :]
        best_i[b0:b0 + _SR, :] = jnp.where(upd, a_t + i * _BK, bi)

    @pl.when(i == pl.num_programs(0) - 1)
    def _():
        idx_out[...] = best_i[...]


def _search(q, db):
    return pl.pallas_call(
        _search_body,
        grid=(_K // _BK,),
        in_specs=[
            pl.BlockSpec((_B, _D), lambda i: (0, 0)),
            pl.BlockSpec((_BK, _D), lambda i: (i, 0)),
        ],
        out_specs=pl.BlockSpec((_B, 1), lambda i: (0, 0)),
        out_shape=jax.ShapeDtypeStruct((_B, 1), jnp.int32),
        scratch_shapes=[
            pltpu.VMEM((_B, 1), jnp.float32),
            pltpu.VMEM((_B, 1), jnp.int32),
        ],
    )(q, db)


_NC = 2
_NS = 16
_NW = _NC * _NS
_BPW = _B // _NW


def _sc_gather_body(db_hbm, idx_hbm, out_hbm, idx_v, rows_v, sem):
    wid = lax.axis_index("s") * _NC + lax.axis_index("c")
    base = wid * _BPW
    pltpu.sync_copy(idx_hbm.at[pl.ds(base, _BPW)], idx_v)
    pltpu.async_copy(db_hbm.at[idx_v], rows_v, sem).wait()
    pltpu.sync_copy(rows_v, out_hbm.at[pl.ds(base, _BPW)])


def _sc_gather(db, idx):
    k = functools.partial(
        pl.kernel,
        out_type=jax.ShapeDtypeStruct((_B, _D), jnp.float32),
        mesh=plsc.VectorSubcoreMesh(core_axis_name="c", subcore_axis_name="s"),
        scratch_types=[
            pltpu.VMEM((_BPW,), jnp.int32),
            pltpu.VMEM((_BPW, _D), jnp.float32),
            pltpu.SemaphoreType.DMA,
        ],
    )(_sc_gather_body)
    return k(db, idx)


def _norm_body(x_ref, o_ref):
    x = x_ref[...]
    n = jnp.sqrt(jnp.sum(x * x, axis=1, keepdims=True))
    o_ref[...] = x / jnp.maximum(n, 1e-12)


def _normalize(x):
    return pl.pallas_call(
        _norm_body,
        out_shape=jax.ShapeDtypeStruct((_B, _D), jnp.float32),
    )(x)


def kernel(pos_emb, neg_emb, db_keys, W1, b1, W2, b2, W3, b3, W4, b4):
    out = _mlp(pos_emb, neg_emb,
               W1, b1.reshape(1, -1), W2, b2.reshape(1, -1),
               W3, b3.reshape(1, -1), W4, b4.reshape(1, -1))
    idx = _search(out, db_keys)
    retrieved = _sc_gather(db_keys, idx.reshape(_B))
    search_out = _normalize(retrieved)
    return (out, search_out)


# speedup vs baseline: 1.0070x; 1.0010x over previous
import functools

import jax
import jax.numpy as jnp
from jax import lax
from jax.experimental import pallas as pl
from jax.experimental.pallas import tpu as pltpu
from jax.experimental.pallas import tpu_sc as plsc

_B = 1024
_D = 768
_K = 100000
_BK = 5000
_NFC = _BK // 128
_SR = 32


def _mlp_body(pos, neg, w1, b1, w2, b2, w3, b3, w4, b4, out):
    x = jnp.concatenate([pos[...], neg[...]], axis=1)
    h = jnp.maximum(jnp.dot(x, w1[...], preferred_element_type=jnp.float32) + b1[...], 0.0)
    h = jnp.maximum(jnp.dot(h, w2[...], preferred_element_type=jnp.float32) + b2[...], 0.0)
    h = jnp.maximum(jnp.dot(h, w3[...], preferred_element_type=jnp.float32) + b3[...], 0.0)
    o = jnp.dot(h, w4[...], preferred_element_type=jnp.float32) + b4[...]
    n = jnp.sqrt(jnp.sum(o * o, axis=1, keepdims=True))
    out[...] = o / jnp.maximum(n, 1e-12)


def _mlp(pos, neg, w1, b1, w2, b2, w3, b3, w4, b4):
    return pl.pallas_call(
        _mlp_body,
        out_shape=jax.ShapeDtypeStruct((_B, _D), jnp.float32),
    )(pos, neg, w1, b1, w2, b2, w3, b3, w4, b4)


def _search_body(q, db, idx_out, best_v, best_i):
    i = pl.program_id(0)

    @pl.when(i == 0)
    def _():
        best_v[...] = jnp.full((_B, 1), -jnp.inf, jnp.float32)
        best_i[...] = jnp.zeros((_B, 1), jnp.int32)

    d = db[...]
    dsq = d * d
    n2 = lax.dot_general(dsq, jnp.ones((1, _D), jnp.float32), (((1,), (1,)), ((), ())),
                         preferred_element_type=jnp.float32)
    rinv = 1.0 / jnp.maximum(jnp.sqrt(n2), 1e-12)
    dn = d * rinv
    s = lax.dot_general(q[...], dn, (((1,), (1,)), ((), ())),
                        preferred_element_type=jnp.float32)

    lane128 = lax.broadcasted_iota(jnp.int32, (_SR, 128), 1)
    lane_tail = lax.broadcasted_iota(jnp.int32, (_SR, _BK - _NFC * 128), 1)

    def _comb(left, right):
        ma, ca = left
        mb, cb = right
        upd = mb > ma
        return jnp.where(upd, mb, ma), jnp.where(upd, cb, ca)

    for rt in range(_B // _SR):
        b0 = rt * _SR
        stack = []
        for c in range(_NFC):
            node = (s[b0:b0 + _SR, c * 128:(c + 1) * 128],
                    jnp.full((_SR, 128), c, jnp.int32))
            lvl = 0
            while stack and stack[-1][0] == lvl:
                _, ma, ca = stack.pop()
                node = _comb((ma, ca), node)
                lvl += 1
            stack.append((lvl, node[0], node[1]))
        node = (stack[-1][1], stack[-1][2])
        for ent in reversed(stack[:-1]):
            node = _comb((ent[1], ent[2]), node)
        tm, tc = node
        pm = s[b0:b0 + _SR, _NFC * 128:_BK]
        m_t = jnp.maximum(jnp.max(tm, axis=1, keepdims=True),
                          jnp.max(pm, axis=1, keepdims=True))
        a1 = jnp.min(jnp.where(tm == m_t, tc * 128 + lane128, _K),
                     axis=1, keepdims=True)
        a2 = jnp.min(jnp.where(pm == m_t, _NFC * 128 + lane_tail, _K),
                     axis=1, keepdims=True)
        a_t = jnp.minimum(a1, a2)
        bv = best_v[b0:b0 + _SR, :]
        upd = m_t > bv
        best_v[b0:b0 + _SR, :] = jnp.where(upd, m_t, bv)
        bi = best_i[b0:b0 + _SR, :]
        best_i[b0:b0 + _SR, :] = jnp.where(upd, a_t + i * _BK, bi)

    @pl.when(i == pl.num_programs(0) - 1)
    def _():
        idx_out[...] = best_i[...]


def _search(q, db):
    return pl.pallas_call(
        _search_body,
        grid=(_K // _BK,),
        in_specs=[
            pl.BlockSpec((_B, _D), lambda i: (0, 0)),
            pl.BlockSpec((_BK, _D), lambda i: (i, 0)),
        ],
        out_specs=pl.BlockSpec((_B, 1), lambda i: (0, 0)),
        out_shape=jax.ShapeDtypeStruct((_B, 1), jnp.int32),
        scratch_shapes=[
            pltpu.VMEM((_B, 1), jnp.float32),
            pltpu.VMEM((_B, 1), jnp.int32),
        ],
    )(q, db)


_NC = 2
_NS = 16
_NW = _NC * _NS
_BPW = _B // _NW


def _sc_gather_body(db_hbm, idx_hbm, out_hbm, idx_v, rows_v, sem):
    wid = lax.axis_index("s") * _NC + lax.axis_index("c")
    base = wid * _BPW
    pltpu.sync_copy(idx_hbm.at[pl.ds(base, _BPW)], idx_v)
    pltpu.async_copy(db_hbm.at[idx_v], rows_v, sem).wait()
    pltpu.sync_copy(rows_v, out_hbm.at[pl.ds(base, _BPW)])


def _sc_gather(db, idx):
    k = functools.partial(
        pl.kernel,
        out_type=jax.ShapeDtypeStruct((_B, _D), jnp.float32),
        mesh=plsc.VectorSubcoreMesh(core_axis_name="c", subcore_axis_name="s"),
        scratch_types=[
            pltpu.VMEM((_BPW,), jnp.int32),
            pltpu.VMEM((_BPW, _D), jnp.float32),
            pltpu.SemaphoreType.DMA,
        ],
    )(_sc_gather_body)
    return k(db, idx)


def _norm_body(x_ref, o_ref):
    x = x_ref[...]
    n = jnp.sqrt(jnp.sum(x * x, axis=1, keepdims=True))
    o_ref[...] = x / jnp.maximum(n, 1e-12)


def _normalize(x):
    return pl.pallas_call(
        _norm_body,
        out_shape=jax.ShapeDtypeStruct((_B, _D), jnp.float32),
    )(x)


def kernel(pos_emb, neg_emb, db_keys, W1, b1, W2, b2, W3, b3, W4, b4):
    out = _mlp(pos_emb, neg_emb,
               W1, b1.reshape(1, -1), W2, b2.reshape(1, -1),
               W3, b3.reshape(1, -1), W4, b4.reshape(1, -1))
    idx = _search(out, db_keys)
    retrieved = _sc_gather(db_keys, idx.reshape(_B))
    search_out = _normalize(retrieved)
    return (out, search_out)
